# initial kernel scaffold (unmeasured)
import jax
import jax.numpy as jnp
from jax import lax
from jax.experimental import pallas as pl
from jax.experimental.pallas import tpu as pltpu

N_DEV = 4
B, SQ, SKV, H, D = 2, 512, 512, 8, 64
DM = 768
HD = H * D
SKV_G = N_DEV * SKV
NEG = -1e9


def kernel(x, Wq, K_ext, V_ext, Wo):
    def body(x_ref, wq_ref, k_ref, v_ref, wo_ref, out_ref,
             kt_all, vt_all, scores_ref, ctx_ref,
             ksend, krecv, vsend, vrecv):
        my = lax.axis_index("i")
        left = jnp.mod(my + N_DEV - 1, N_DEV)
        right = jnp.mod(my + 1, N_DEV)

        barrier_sem = pltpu.get_barrier_semaphore()
        for nbr in (left, right):
            pl.semaphore_signal(barrier_sem, inc=1, device_id=(nbr,),
                                device_id_type=pl.DeviceIdType.MESH)
        pl.semaphore_wait(barrier_sem, 2)

        for b in range(B):
            for h in range(H):
                kt_all[my, b, h] = k_ref[b, :, h, :]
                vt_all[my, b, h] = v_ref[b, :, h, :]

        for hop in range(N_DEV - 1):
            o_send = jnp.mod(my - hop + N_DEV, N_DEV)
            o_recv = jnp.mod(my - hop - 1 + N_DEV, N_DEV)
            k_rdma = pltpu.make_async_remote_copy(
                src_ref=kt_all.at[o_send], dst_ref=kt_all.at[o_send],
                send_sem=ksend.at[hop], recv_sem=krecv.at[hop],
                device_id=(right,), device_id_type=pl.DeviceIdType.MESH,
            )
            v_rdma = pltpu.make_async_remote_copy(
                src_ref=vt_all.at[o_send], dst_ref=vt_all.at[o_send],
                send_sem=vsend.at[hop], recv_sem=vrecv.at[hop],
                device_id=(right,), device_id_type=pl.DeviceIdType.MESH,
            )
            k_rdma.start()
            v_rdma.start()
            k_rdma.wait()
            v_rdma.wait()
            del o_recv

        q0 = my * SQ
        for b in range(B):
            for h in range(H):
                q_bh = jnp.dot(x_ref[b], wq_ref[:, h * D:(h + 1) * D],
                               preferred_element_type=jnp.float32)
                qi = q0 + lax.broadcasted_iota(jnp.int32, (SQ, SKV), 0)
                for o in range(N_DEV):
                    s = lax.dot_general(
                        q_bh, kt_all[o, b, h],
                        (((1,), (1,)), ((), ())),
                        preferred_element_type=jnp.float32) * 0.125
                    ki = o * SKV + lax.broadcasted_iota(jnp.int32, (SQ, SKV), 1)
                    mask = (jnp.abs(qi - ki) <= 128) | (ki < 32) | (qi < 32)
                    scores_ref[:, o * SKV:(o + 1) * SKV] = jnp.where(mask, s, NEG)
                m = jnp.max(scores_ref[...], axis=1, keepdims=True)
                l = jnp.zeros((SQ, 1), jnp.float32)
                acc = jnp.zeros((SQ, D), jnp.float32)
                for o in range(N_DEV):
                    w = jnp.exp(scores_ref[:, o * SKV:(o + 1) * SKV] - m)
                    l = l + jnp.sum(w, axis=1, keepdims=True)
                    acc = acc + jnp.dot(w, vt_all[o, b, h],
                                        preferred_element_type=jnp.float32)
                ctx_ref[b, :, h * D:(h + 1) * D] = acc / l

        for b in range(B):
            out_ref[b] = jnp.dot(ctx_ref[b], wo_ref[...],
                                 preferred_element_type=jnp.float32)

    return pl.pallas_call(
        body,
        out_shape=jax.ShapeDtypeStruct((B, SQ, DM), jnp.float32),
        in_specs=[pl.BlockSpec(memory_space=pltpu.VMEM)] * 5,
        out_specs=pl.BlockSpec(memory_space=pltpu.VMEM),
        scratch_shapes=[
            pltpu.VMEM((N_DEV, B, H, SKV, D), jnp.float32),
            pltpu.VMEM((N_DEV, B, H, SKV, D), jnp.float32),
            pltpu.VMEM((SQ, SKV_G), jnp.float32),
            pltpu.VMEM((B, SQ, HD), jnp.float32),
            pltpu.SemaphoreType.DMA((N_DEV - 1,)),
            pltpu.SemaphoreType.DMA((N_DEV - 1,)),
            pltpu.SemaphoreType.DMA((N_DEV - 1,)),
            pltpu.SemaphoreType.DMA((N_DEV - 1,)),
        ],
        compiler_params=pltpu.CompilerParams(collective_id=0),
    )(x, Wq, K_ext, V_ext, Wo)


# baseline (device time: 202762 ns/iter reference)
import jax
import jax.numpy as jnp
from jax import lax
from jax.experimental import pallas as pl
from jax.experimental.pallas import tpu as pltpu

N_DEV = 4
B, SQ, SKV, H, D = 2, 512, 512, 8, 64
BH = B * H
DM = 768
HD = H * D
NEG = -1e9


def kernel(x, Wq, K_ext, V_ext, Wo):
    def body(x_ref, wq_ref, k_ref, v_ref, wo_ref, out_ref,
             kt_all, vt_all, q_all, ctx_all, scores_ref,
             ksend, krecv, vsend, vrecv):
        my = lax.axis_index("i")
        left = jnp.mod(my + N_DEV - 1, N_DEV)
        right = jnp.mod(my + 1, N_DEV)

        barrier_sem = pltpu.get_barrier_semaphore()
        for nbr in (left, right):
            pl.semaphore_signal(barrier_sem, inc=1, device_id=(nbr,),
                                device_id_type=pl.DeviceIdType.MESH)
        pl.semaphore_wait(barrier_sem, 2)

        for b in range(B):
            for h in range(H):
                kt_all[my, b * H + h] = k_ref[b, :, h, :].T
                vt_all[my, b * H + h] = v_ref[b, :, h, :].T

        for b in range(B):
            q_b = jnp.dot(x_ref[b], wq_ref[...],
                          preferred_element_type=jnp.float32)
            for h in range(H):
                q_all[b * H + h] = q_b[:, h * D:(h + 1) * D]

        for hop in range(N_DEV - 1):
            o_send = jnp.mod(my - hop + N_DEV, N_DEV)
            k_rdma = pltpu.make_async_remote_copy(
                src_ref=kt_all.at[o_send], dst_ref=kt_all.at[o_send],
                send_sem=ksend.at[hop], recv_sem=krecv.at[hop],
                device_id=(right,), device_id_type=pl.DeviceIdType.MESH,
            )
            v_rdma = pltpu.make_async_remote_copy(
                src_ref=vt_all.at[o_send], dst_ref=vt_all.at[o_send],
                send_sem=vsend.at[hop], recv_sem=vrecv.at[hop],
                device_id=(right,), device_id_type=pl.DeviceIdType.MESH,
            )
            k_rdma.start()
            v_rdma.start()
            k_rdma.wait()
            v_rdma.wait()

        q0 = my * SQ

        def bh_step(bh, _):
            q = q_all[bh]
            qi = q0 + lax.broadcasted_iota(jnp.int32, (SQ, SKV), 0)
            m = jnp.full((SQ, 1), NEG, jnp.float32)
            for o in range(N_DEV):
                s = jnp.dot(q, kt_all[o, bh],
                            preferred_element_type=jnp.float32) * 0.125
                ki = o * SKV + lax.broadcasted_iota(jnp.int32, (SQ, SKV), 1)
                mask = (jnp.abs(qi - ki) <= 128) | (ki < 32) | (qi < 32)
                s = jnp.where(mask, s, NEG)
                scores_ref[:, o * SKV:(o + 1) * SKV] = s
                m = jnp.maximum(m, jnp.max(s, axis=1, keepdims=True))
            l = jnp.zeros((SQ, 1), jnp.float32)
            acc = jnp.zeros((SQ, D), jnp.float32)
            for o in range(N_DEV):
                w = jnp.exp(scores_ref[:, o * SKV:(o + 1) * SKV] - m)
                l = l + jnp.sum(w, axis=1, keepdims=True)
                acc = acc + lax.dot_general(
                    w, vt_all[o, bh], (((1,), (1,)), ((), ())),
                    preferred_element_type=jnp.float32)
            ctx_all[bh] = acc / l
            return 0

        lax.fori_loop(0, BH, bh_step, 0)

        for b in range(B):
            acc = jnp.zeros((SQ, DM), jnp.float32)
            for h in range(H):
                acc = acc + jnp.dot(ctx_all[b * H + h],
                                    wo_ref[h * D:(h + 1) * D, :],
                                    preferred_element_type=jnp.float32)
            out_ref[b] = acc

    return pl.pallas_call(
        body,
        out_shape=jax.ShapeDtypeStruct((B, SQ, DM), jnp.float32),
        in_specs=[pl.BlockSpec(memory_space=pltpu.VMEM)] * 5,
        out_specs=pl.BlockSpec(memory_space=pltpu.VMEM),
        scratch_shapes=[
            pltpu.VMEM((N_DEV, BH, D, SKV), jnp.float32),
            pltpu.VMEM((N_DEV, BH, D, SKV), jnp.float32),
            pltpu.VMEM((BH, SQ, D), jnp.float32),
            pltpu.VMEM((BH, SQ, D), jnp.float32),
            pltpu.VMEM((SQ, N_DEV * SKV), jnp.float32),
            pltpu.SemaphoreType.DMA((N_DEV - 1,)),
            pltpu.SemaphoreType.DMA((N_DEV - 1,)),
            pltpu.SemaphoreType.DMA((N_DEV - 1,)),
            pltpu.SemaphoreType.DMA((N_DEV - 1,)),
        ],
        compiler_params=pltpu.CompilerParams(collective_id=0),
    )(x, Wq, K_ext, V_ext, Wo)


# device time: 83343 ns/iter; 2.4329x vs baseline; 2.4329x over previous
import jax
import jax.numpy as jnp
from jax import lax
from jax.experimental import pallas as pl
from jax.experimental.pallas import tpu as pltpu

N_DEV = 4
B, SQ, SKV, H, D = 2, 512, 512, 8, 64
BH = B * H
DM = 768
HD = H * D
HALO = 128
G = 32
NEG = -1e9


def kernel(x, Wq, K_ext, V_ext, Wo):
    def body(x_ref, wq_ref, k_ref, v_ref, wo_ref, out_ref,
             kt_own, vt_own, q_all, ctx_all,
             khalo_l, vhalo_l, khalo_r, vhalo_r,
             kglob, vglob, kg_stage, vg_stage,
             q32_buf, c32_send, stats_send, c32_recv, stats_recv,
             hsend, hrecv, gsend, grecv, qsend, qrecv, psend, precv):
        my = lax.axis_index("i")
        left = my - 1
        right = my + 1
        has_left = my > 0
        has_right = my < N_DEV - 1
        is0 = my == 0

        vhalo_l[...] = jnp.zeros_like(vhalo_l)
        vhalo_r[...] = jnp.zeros_like(vhalo_r)
        vglob[...] = jnp.zeros_like(vglob)

        n_partners = (has_left.astype(jnp.int32) + has_right.astype(jnp.int32)
                      + jnp.where(is0, 2, 0) + jnp.where(my >= 2, 1, 0))
        barrier_sem = pltpu.get_barrier_semaphore()

        def _signal(sem, dev):
            pl.semaphore_signal(sem, inc=1, device_id=(dev,),
                                device_id_type=pl.DeviceIdType.MESH)

        @pl.when(has_left)
        def _():
            _signal(barrier_sem, left)

        @pl.when(has_right)
        def _():
            _signal(barrier_sem, right)

        @pl.when(is0)
        def _():
            _signal(barrier_sem, 2)
            _signal(barrier_sem, 3)

        @pl.when(my >= 2)
        def _():
            _signal(barrier_sem, 0)

        pl.semaphore_wait(barrier_sem, n_partners)

        for b in range(B):
            for h in range(H):
                kt_own[b * H + h] = k_ref[b, :, h, :].T
                vt_own[b * H + h] = v_ref[b, :, h, :].T
        for b in range(B):
            q_b = jnp.dot(x_ref[b], wq_ref[...],
                          preferred_element_type=jnp.float32)
            for h in range(H):
                q_all[b * H + h] = q_b[:, h * D:(h + 1) * D]

        @pl.when(is0)
        def _():
            kg_stage[...] = kt_own[:, :, 0:G]
            vg_stage[...] = vt_own[:, :, 0:G]
            q32_buf[...] = q_all[:, 0:G, :]

        halo_r_k = pltpu.make_async_remote_copy(
            src_ref=kt_own.at[:, :, SKV - HALO:SKV], dst_ref=khalo_l,
            send_sem=hsend.at[0], recv_sem=hrecv.at[0],
            device_id=(right,), device_id_type=pl.DeviceIdType.MESH)
        halo_r_v = pltpu.make_async_remote_copy(
            src_ref=vt_own.at[:, :, SKV - HALO:SKV], dst_ref=vhalo_l,
            send_sem=hsend.at[1], recv_sem=hrecv.at[1],
            device_id=(right,), device_id_type=pl.DeviceIdType.MESH)
        halo_l_k = pltpu.make_async_remote_copy(
            src_ref=kt_own.at[:, :, 0:HALO], dst_ref=khalo_r,
            send_sem=hsend.at[2], recv_sem=hrecv.at[2],
            device_id=(left,), device_id_type=pl.DeviceIdType.MESH)
        halo_l_v = pltpu.make_async_remote_copy(
            src_ref=vt_own.at[:, :, 0:HALO], dst_ref=vhalo_r,
            send_sem=hsend.at[3], recv_sem=hrecv.at[3],
            device_id=(left,), device_id_type=pl.DeviceIdType.MESH)
        glob_k = [pltpu.make_async_remote_copy(
            src_ref=kg_stage, dst_ref=kglob,
            send_sem=gsend.at[t - 1], recv_sem=grecv.at[0],
            device_id=(t,), device_id_type=pl.DeviceIdType.MESH)
            for t in (1, 2, 3)]
        glob_v = [pltpu.make_async_remote_copy(
            src_ref=vg_stage, dst_ref=vglob,
            send_sem=gsend.at[t + 2], recv_sem=grecv.at[1],
            device_id=(t,), device_id_type=pl.DeviceIdType.MESH)
            for t in (1, 2, 3)]
        q32_bcast = [pltpu.make_async_remote_copy(
            src_ref=q32_buf, dst_ref=q32_buf,
            send_sem=qsend.at[t - 1], recv_sem=qrecv.at[0],
            device_id=(t,), device_id_type=pl.DeviceIdType.MESH)
            for t in (1, 2, 3)]
        part_c = pltpu.make_async_remote_copy(
            src_ref=c32_send, dst_ref=c32_recv.at[my - 1],
            send_sem=psend.at[0], recv_sem=precv.at[my - 1],
            device_id=(0,), device_id_type=pl.DeviceIdType.MESH)
        part_s = pltpu.make_async_remote_copy(
            src_ref=stats_send, dst_ref=stats_recv.at[my - 1],
            send_sem=psend.at[1], recv_sem=precv.at[my + 2],
            device_id=(0,), device_id_type=pl.DeviceIdType.MESH)
        part_c_recv = [pltpu.make_async_remote_copy(
            src_ref=c32_send, dst_ref=c32_recv.at[r - 1],
            send_sem=psend.at[0], recv_sem=precv.at[r - 1],
            device_id=(0,), device_id_type=pl.DeviceIdType.MESH)
            for r in (1, 2, 3)]
        part_s_recv = [pltpu.make_async_remote_copy(
            src_ref=stats_send, dst_ref=stats_recv.at[r - 1],
            send_sem=psend.at[1], recv_sem=precv.at[r + 2],
            device_id=(0,), device_id_type=pl.DeviceIdType.MESH)
            for r in (1, 2, 3)]

        @pl.when(has_right)
        def _():
            halo_r_k.start()
            halo_r_v.start()

        @pl.when(has_left)
        def _():
            halo_l_k.start()
            halo_l_v.start()

        @pl.when(is0)
        def _():
            for t in range(3):
                glob_k[t].start()
                glob_v[t].start()
                q32_bcast[t].start()

        @pl.when(has_left)
        def _():
            halo_r_k.wait_recv()
            halo_r_v.wait_recv()

        @pl.when(has_right)
        def _():
            halo_l_k.wait_recv()
            halo_l_v.wait_recv()

        @pl.when(~is0)
        def _():
            glob_k[0].wait_recv()
            glob_v[0].wait_recv()
            q32_bcast[0].wait_recv()

        q0 = my * SQ

        def bh_step(bh, _):
            q = q_all[bh]
            i_own = lax.broadcasted_iota(jnp.int32, (SQ, SKV), 0)
            j_own = lax.broadcasted_iota(jnp.int32, (SQ, SKV), 1)
            i_h = lax.broadcasted_iota(jnp.int32, (SQ, HALO), 0)
            j_h = lax.broadcasted_iota(jnp.int32, (SQ, HALO), 1)
            j_g = lax.broadcasted_iota(jnp.int32, (SQ, G), 1)

            m = jnp.full((SQ, 1), NEG, jnp.float32)
            l = jnp.zeros((SQ, 1), jnp.float32)
            acc = jnp.zeros((SQ, D), jnp.float32)

            def block(kt, vt, mask, m, l, acc):
                s = jnp.dot(q, kt, preferred_element_type=jnp.float32) * 0.125
                s = jnp.where(mask, s, NEG)
                m_new = jnp.maximum(m, jnp.max(s, axis=1, keepdims=True))
                alpha = jnp.exp(m - m_new)
                w = jnp.exp(s - m_new)
                l = l * alpha + jnp.sum(w, axis=1, keepdims=True)
                acc = acc * alpha + lax.dot_general(
                    w, vt, (((1,), (1,)), ((), ())),
                    preferred_element_type=jnp.float32)
                return m_new, l, acc

            mask_own = ((jnp.abs(i_own - j_own) <= HALO)
                        | (is0 & (j_own < G)) | (is0 & (i_own < G)))
            m, l, acc = block(kt_own[bh], vt_own[bh], mask_own, m, l, acc)
            mask_l = has_left & (i_h <= j_h)
            m, l, acc = block(khalo_l[bh], vhalo_l[bh], mask_l, m, l, acc)
            mask_r = has_right & (i_h >= (SQ - HALO) + j_h)
            m, l, acc = block(khalo_r[bh], vhalo_r[bh], mask_r, m, l, acc)
            mask_g = ~is0 & (j_g < G)
            m, l, acc = block(kglob[bh], vglob[bh], mask_g, m, l, acc)

            ctx_all[bh] = acc / l

            s32 = jnp.dot(q32_buf[bh], kt_own[bh],
                          preferred_element_type=jnp.float32) * 0.125
            m32 = jnp.max(s32, axis=1, keepdims=True)
            w32 = jnp.exp(s32 - m32)
            l32 = jnp.sum(w32, axis=1, keepdims=True)
            c32 = lax.dot_general(w32, vt_own[bh], (((1,), (1,)), ((), ())),
                                  preferred_element_type=jnp.float32)
            c32_send[bh] = c32
            stats_send[bh, :, 0:1] = m32
            stats_send[bh, :, 1:2] = l32
            return 0

        lax.fori_loop(0, BH, bh_step, 0)

        @pl.when(~is0)
        def _():
            part_c.start()
            part_s.start()

        @pl.when(is0)
        def _():
            for r in range(3):
                part_c_recv[r].wait_recv()
                part_s_recv[r].wait_recv()
            ms = [stats_send[:, :, 0:1]] + [stats_recv[r, :, :, 0:1]
                                            for r in range(3)]
            ls = [stats_send[:, :, 1:2]] + [stats_recv[r, :, :, 1:2]
                                            for r in range(3)]
            cs = [c32_send[...]] + [c32_recv[r] for r in range(3)]
            M = jnp.maximum(jnp.maximum(ms[0], ms[1]),
                            jnp.maximum(ms[2], ms[3]))
            L = jnp.zeros_like(ls[0])
            C = jnp.zeros_like(cs[0])
            for s in range(4):
                a = jnp.exp(ms[s] - M)
                L = L + a * ls[s]
                C = C + a * cs[s]
            ctx_all[:, 0:G, :] = C / L

        for b in range(B):
            acc = jnp.zeros((SQ, DM), jnp.float32)
            for h in range(H):
                acc = acc + jnp.dot(ctx_all[b * H + h],
                                    wo_ref[h * D:(h + 1) * D, :],
                                    preferred_element_type=jnp.float32)
            out_ref[b] = acc

        @pl.when(has_right)
        def _():
            halo_r_k.wait_send()
            halo_r_v.wait_send()

        @pl.when(has_left)
        def _():
            halo_l_k.wait_send()
            halo_l_v.wait_send()

        @pl.when(is0)
        def _():
            for t in range(3):
                glob_k[t].wait_send()
                glob_v[t].wait_send()
                q32_bcast[t].wait_send()

        @pl.when(~is0)
        def _():
            part_c.wait_send()
            part_s.wait_send()

        def _second(second_barrier):
            @pl.when(has_left)
            def _():
                _signal(second_barrier, left)

            @pl.when(has_right)
            def _():
                _signal(second_barrier, right)

            @pl.when(is0)
            def _():
                _signal(second_barrier, 2)
                _signal(second_barrier, 3)

            @pl.when(my >= 2)
            def _():
                _signal(second_barrier, 0)

            pl.semaphore_wait(second_barrier, n_partners)

        pl.run_scoped(_second, second_barrier=pltpu.SemaphoreType.REGULAR)

    return pl.pallas_call(
        body,
        out_shape=jax.ShapeDtypeStruct((B, SQ, DM), jnp.float32),
        in_specs=[pl.BlockSpec(memory_space=pltpu.VMEM)] * 5,
        out_specs=pl.BlockSpec(memory_space=pltpu.VMEM),
        scratch_shapes=[
            pltpu.VMEM((BH, D, SKV), jnp.float32),
            pltpu.VMEM((BH, D, SKV), jnp.float32),
            pltpu.VMEM((BH, SQ, D), jnp.float32),
            pltpu.VMEM((BH, SQ, D), jnp.float32),
            pltpu.VMEM((BH, D, HALO), jnp.float32),
            pltpu.VMEM((BH, D, HALO), jnp.float32),
            pltpu.VMEM((BH, D, HALO), jnp.float32),
            pltpu.VMEM((BH, D, HALO), jnp.float32),
            pltpu.VMEM((BH, D, G), jnp.float32),
            pltpu.VMEM((BH, D, G), jnp.float32),
            pltpu.VMEM((BH, D, G), jnp.float32),
            pltpu.VMEM((BH, D, G), jnp.float32),
            pltpu.VMEM((BH, G, D), jnp.float32),
            pltpu.VMEM((BH, G, D), jnp.float32),
            pltpu.VMEM((BH, G, 2), jnp.float32),
            pltpu.VMEM((3, BH, G, D), jnp.float32),
            pltpu.VMEM((3, BH, G, 2), jnp.float32),
            pltpu.SemaphoreType.DMA((4,)),
            pltpu.SemaphoreType.DMA((4,)),
            pltpu.SemaphoreType.DMA((6,)),
            pltpu.SemaphoreType.DMA((2,)),
            pltpu.SemaphoreType.DMA((3,)),
            pltpu.SemaphoreType.DMA((1,)),
            pltpu.SemaphoreType.DMA((2,)),
            pltpu.SemaphoreType.DMA((6,)),
        ],
        compiler_params=pltpu.CompilerParams(collective_id=0),
    )(x, Wq, K_ext, V_ext, Wo)


# device time: 63873 ns/iter; 3.1745x vs baseline; 1.3048x over previous
import jax
import jax.numpy as jnp
from jax import lax
from jax.experimental import pallas as pl
from jax.experimental.pallas import tpu as pltpu

N_DEV = 4
B, SQ, SKV, H, D = 2, 512, 512, 8, 64
DV = D + 1
BH = B * H
DM = 768
HD = H * D
HALO = 128
G = 32
T = 256
BAND = 384
NEG = -1e9


def kernel(x, Wq, K_ext, V_ext, Wo):
    bf16 = jnp.bfloat16
    k2 = jnp.transpose(K_ext, (0, 2, 1, 3)).reshape(BH, SKV, D).astype(bf16)
    v2 = jnp.transpose(V_ext, (0, 2, 1, 3)).reshape(BH, SKV, D).astype(bf16)
    v2 = jnp.concatenate([v2, jnp.ones((BH, SKV, 1), bf16)], axis=2)
    x16 = x.astype(bf16)
    wq16 = (Wq * 0.125).astype(bf16)
    wo16 = Wo.astype(bf16)

    def body(x_ref, wq_ref, k_ref, v_ref, wo_ref, out_ref,
             q_all, ctx_all,
             khalo_l, vhalo_l, khalo_r, vhalo_r, kglob, vglob,
             bias_band, bias_halo, bias_g,
             q32_buf, c32_send, stats_send, c32_recv, stats_recv,
             hsend, hrecv, gsend, grecv, qsend, qrecv, psend, precv):
        my = lax.axis_index("i")
        left = my - 1
        right = my + 1
        has_left = my > 0
        has_right = my < N_DEV - 1
        is0 = my == 0

        khalo_l[...] = jnp.zeros_like(khalo_l)
        vhalo_l[...] = jnp.zeros_like(vhalo_l)
        khalo_r[...] = jnp.zeros_like(khalo_r)
        vhalo_r[...] = jnp.zeros_like(vhalo_r)

        n_partners = (has_left.astype(jnp.int32) + has_right.astype(jnp.int32)
                      + jnp.where(is0, 2, 0) + jnp.where(my >= 2, 1, 0))
        barrier_sem = pltpu.get_barrier_semaphore()

        def _signal(sem, dev):
            pl.semaphore_signal(sem, inc=1, device_id=(dev,),
                                device_id_type=pl.DeviceIdType.MESH)

        @pl.when(has_left)
        def _():
            _signal(barrier_sem, left)

        @pl.when(has_right)
        def _():
            _signal(barrier_sem, right)

        @pl.when(is0)
        def _():
            _signal(barrier_sem, 2)
            _signal(barrier_sem, 3)

        @pl.when(my >= 2)
        def _():
            _signal(barrier_sem, 0)

        pl.semaphore_wait(barrier_sem, n_partners)

        halo_r_k = pltpu.make_async_remote_copy(
            src_ref=k_ref.at[:, SKV - HALO:SKV, :], dst_ref=khalo_l,
            send_sem=hsend.at[0], recv_sem=hrecv.at[0],
            device_id=(right,), device_id_type=pl.DeviceIdType.MESH)
        halo_r_v = pltpu.make_async_remote_copy(
            src_ref=v_ref.at[:, SKV - HALO:SKV, :], dst_ref=vhalo_l,
            send_sem=hsend.at[1], recv_sem=hrecv.at[1],
            device_id=(right,), device_id_type=pl.DeviceIdType.MESH)
        halo_l_k = pltpu.make_async_remote_copy(
            src_ref=k_ref.at[:, 0:HALO, :], dst_ref=khalo_r,
            send_sem=hsend.at[2], recv_sem=hrecv.at[2],
            device_id=(left,), device_id_type=pl.DeviceIdType.MESH)
        halo_l_v = pltpu.make_async_remote_copy(
            src_ref=v_ref.at[:, 0:HALO, :], dst_ref=vhalo_r,
            send_sem=hsend.at[3], recv_sem=hrecv.at[3],
            device_id=(left,), device_id_type=pl.DeviceIdType.MESH)
        glob_k = [pltpu.make_async_remote_copy(
            src_ref=k_ref.at[:, 0:G, :], dst_ref=kglob,
            send_sem=gsend.at[t - 1], recv_sem=grecv.at[0],
            device_id=(t,), device_id_type=pl.DeviceIdType.MESH)
            for t in (1, 2, 3)]
        glob_v = [pltpu.make_async_remote_copy(
            src_ref=v_ref.at[:, 0:G, :], dst_ref=vglob,
            send_sem=gsend.at[t + 2], recv_sem=grecv.at[1],
            device_id=(t,), device_id_type=pl.DeviceIdType.MESH)
            for t in (1, 2, 3)]
        q32_bcast = [pltpu.make_async_remote_copy(
            src_ref=q32_buf, dst_ref=q32_buf,
            send_sem=qsend.at[t - 1], recv_sem=qrecv.at[0],
            device_id=(t,), device_id_type=pl.DeviceIdType.MESH)
            for t in (1, 2, 3)]
        part_c = pltpu.make_async_remote_copy(
            src_ref=c32_send, dst_ref=c32_recv.at[my - 1],
            send_sem=psend.at[0], recv_sem=precv.at[my - 1],
            device_id=(0,), device_id_type=pl.DeviceIdType.MESH)
        part_s = pltpu.make_async_remote_copy(
            src_ref=stats_send, dst_ref=stats_recv.at[my - 1],
            send_sem=psend.at[1], recv_sem=precv.at[my + 2],
            device_id=(0,), device_id_type=pl.DeviceIdType.MESH)
        part_c_recv = [pltpu.make_async_remote_copy(
            src_ref=c32_send, dst_ref=c32_recv.at[r - 1],
            send_sem=psend.at[0], recv_sem=precv.at[r - 1],
            device_id=(0,), device_id_type=pl.DeviceIdType.MESH)
            for r in (1, 2, 3)]
        part_s_recv = [pltpu.make_async_remote_copy(
            src_ref=stats_send, dst_ref=stats_recv.at[r - 1],
            send_sem=psend.at[1], recv_sem=precv.at[r + 2],
            device_id=(0,), device_id_type=pl.DeviceIdType.MESH)
            for r in (1, 2, 3)]

        @pl.when(has_right)
        def _():
            halo_r_k.start()
            halo_r_v.start()

        @pl.when(has_left)
        def _():
            halo_l_k.start()
            halo_l_v.start()

        @pl.when(is0)
        def _():
            for t in range(3):
                glob_k[t].start()
                glob_v[t].start()
            for b in range(B):
                q32_b = jnp.dot(x_ref[b, 0:G, :], wq_ref[...],
                                preferred_element_type=jnp.float32)
                for h in range(H):
                    q32_buf[b * H + h] = q32_b[:, h * D:(h + 1) * D].astype(
                        jnp.bfloat16)
            for t in range(3):
                q32_bcast[t].start()
            kglob[...] = k_ref[:, 0:G, :]
            vglob[...] = v_ref[:, 0:G, :]

        for b in range(B):
            q_b = jnp.dot(x_ref[b], wq_ref[...],
                          preferred_element_type=jnp.float32)
            for h in range(H):
                q_all[b * H + h] = q_b[:, h * D:(h + 1) * D].astype(
                    jnp.bfloat16)

        ib = lax.broadcasted_iota(jnp.int32, (T, BAND), 0)
        jb = lax.broadcasted_iota(jnp.int32, (T, BAND), 1)
        ih = lax.broadcasted_iota(jnp.int32, (T, HALO), 0)
        jh = lax.broadcasted_iota(jnp.int32, (T, HALO), 1)
        ig = lax.broadcasted_iota(jnp.int32, (T, G), 0)
        jg = lax.broadcasted_iota(jnp.int32, (T, G), 1)
        zero = jnp.float32(0)
        for t in range(2):
            a0 = 0 if t == 0 else SKV - BAND
            bias_band[t] = jnp.where(
                jnp.abs((ib + t * T) - (jb + a0)) <= HALO, zero, NEG)
        bias_halo[0] = jnp.where(has_left & (ih <= jh), zero, NEG)
        bias_halo[1] = jnp.where(has_right & (ih >= HALO + jh), zero, NEG)
        bias_g[...] = jnp.where((~is0) | (ig > HALO + jg), zero, NEG)

        @pl.when(has_left)
        def _():
            halo_r_k.wait_recv()
            halo_r_v.wait_recv()

        @pl.when(has_right)
        def _():
            halo_l_k.wait_recv()
            halo_l_v.wait_recv()

        @pl.when(~is0)
        def _():
            glob_k[0].wait_recv()
            glob_v[0].wait_recv()
            q32_bcast[0].wait_recv()

        def bh_step(bh, _):
            q = q_all[bh]

            for t in range(2):
                q_t = q[t * T:(t + 1) * T, :]
                a0 = 0 if t == 0 else SKV - BAND
                s = lax.dot_general(q_t, k_ref[bh, a0:a0 + BAND, :],
                                    (((1,), (1,)), ((), ())),
                                    preferred_element_type=jnp.float32)
                w = jnp.exp(s + bias_band[t]).astype(jnp.bfloat16)
                acc = lax.dot_general(w, v_ref[bh, a0:a0 + BAND, :],
                                      (((1,), (0,)), ((), ())),
                                      preferred_element_type=jnp.float32)
                kh = khalo_l if t == 0 else khalo_r
                vh = vhalo_l if t == 0 else vhalo_r
                s = lax.dot_general(q_t, kh[bh], (((1,), (1,)), ((), ())),
                                    preferred_element_type=jnp.float32)
                w = jnp.exp(s + bias_halo[t]).astype(jnp.bfloat16)
                acc = acc + lax.dot_general(w, vh[bh], (((1,), (0,)), ((), ())),
                                            preferred_element_type=jnp.float32)
                s = lax.dot_general(q_t, kglob[bh], (((1,), (1,)), ((), ())),
                                    preferred_element_type=jnp.float32)
                if t == 0:
                    s = s + bias_g[...]
                w = jnp.exp(s).astype(jnp.bfloat16)
                acc = acc + lax.dot_general(w, vglob[bh],
                                            (((1,), (0,)), ((), ())),
                                            preferred_element_type=jnp.float32)
                ctx_all[bh, t * T:(t + 1) * T, :] = (
                    acc[:, 0:D] / acc[:, D:DV]).astype(jnp.bfloat16)

            s32 = lax.dot_general(q32_buf[bh], k_ref[bh],
                                  (((1,), (1,)), ((), ())),
                                  preferred_element_type=jnp.float32)
            m32 = jnp.max(s32, axis=1, keepdims=True)
            w32 = jnp.exp(s32 - m32).astype(jnp.bfloat16)
            ce = lax.dot_general(w32, v_ref[bh], (((1,), (0,)), ((), ())),
                                 preferred_element_type=jnp.float32)
            c32_send[bh] = ce[:, 0:D]
            stats_send[bh, :, 0:1] = m32
            stats_send[bh, :, 1:2] = ce[:, D:DV]
            return 0

        lax.fori_loop(0, BH, bh_step, 0)

        @pl.when(~is0)
        def _():
            part_c.start()
            part_s.start()

        @pl.when(is0)
        def _():
            for r in range(3):
                part_c_recv[r].wait_recv()
                part_s_recv[r].wait_recv()
            ms = [stats_send[:, :, 0:1]] + [stats_recv[r, :, :, 0:1]
                                            for r in range(3)]
            ls = [stats_send[:, :, 1:2]] + [stats_recv[r, :, :, 1:2]
                                            for r in range(3)]
            cs = [c32_send[...]] + [c32_recv[r] for r in range(3)]
            M = jnp.maximum(jnp.maximum(ms[0], ms[1]),
                            jnp.maximum(ms[2], ms[3]))
            L = jnp.zeros_like(ls[0])
            C = jnp.zeros_like(cs[0])
            for s in range(4):
                a = jnp.exp(ms[s] - M)
                L = L + a * ls[s]
                C = C + a * cs[s]
            ctx_all[:, 0:G, :] = (C / L).astype(jnp.bfloat16)

        for b in range(B):
            acc = jnp.zeros((SQ, DM), jnp.float32)
            for h in range(H):
                acc = acc + jnp.dot(ctx_all[b * H + h],
                                    wo_ref[h * D:(h + 1) * D, :],
                                    preferred_element_type=jnp.float32)
            out_ref[b] = acc

        @pl.when(has_right)
        def _():
            halo_r_k.wait_send()
            halo_r_v.wait_send()

        @pl.when(has_left)
        def _():
            halo_l_k.wait_send()
            halo_l_v.wait_send()

        @pl.when(is0)
        def _():
            for t in range(3):
                glob_k[t].wait_send()
                glob_v[t].wait_send()
                q32_bcast[t].wait_send()

        @pl.when(~is0)
        def _():
            part_c.wait_send()
            part_s.wait_send()

        def _second(second_barrier):
            @pl.when(has_left)
            def _():
                _signal(second_barrier, left)

            @pl.when(has_right)
            def _():
                _signal(second_barrier, right)

            @pl.when(is0)
            def _():
                _signal(second_barrier, 2)
                _signal(second_barrier, 3)

            @pl.when(my >= 2)
            def _():
                _signal(second_barrier, 0)

            pl.semaphore_wait(second_barrier, n_partners)

        pl.run_scoped(_second, second_barrier=pltpu.SemaphoreType.REGULAR)

    return pl.pallas_call(
        body,
        out_shape=jax.ShapeDtypeStruct((B, SQ, DM), jnp.float32),
        in_specs=[pl.BlockSpec(memory_space=pltpu.VMEM)] * 5,
        out_specs=pl.BlockSpec(memory_space=pltpu.VMEM),
        scratch_shapes=[
            pltpu.VMEM((BH, SQ, D), jnp.bfloat16),
            pltpu.VMEM((BH, SQ, D), jnp.bfloat16),
            pltpu.VMEM((BH, HALO, D), jnp.bfloat16),
            pltpu.VMEM((BH, HALO, DV), jnp.bfloat16),
            pltpu.VMEM((BH, HALO, D), jnp.bfloat16),
            pltpu.VMEM((BH, HALO, DV), jnp.bfloat16),
            pltpu.VMEM((BH, G, D), jnp.bfloat16),
            pltpu.VMEM((BH, G, DV), jnp.bfloat16),
            pltpu.VMEM((2, T, BAND), jnp.float32),
            pltpu.VMEM((2, T, HALO), jnp.float32),
            pltpu.VMEM((T, G), jnp.float32),
            pltpu.VMEM((BH, G, D), jnp.bfloat16),
            pltpu.VMEM((BH, G, D), jnp.float32),
            pltpu.VMEM((BH, G, 2), jnp.float32),
            pltpu.VMEM((3, BH, G, D), jnp.float32),
            pltpu.VMEM((3, BH, G, 2), jnp.float32),
            pltpu.SemaphoreType.DMA((4,)),
            pltpu.SemaphoreType.DMA((4,)),
            pltpu.SemaphoreType.DMA((6,)),
            pltpu.SemaphoreType.DMA((2,)),
            pltpu.SemaphoreType.DMA((3,)),
            pltpu.SemaphoreType.DMA((1,)),
            pltpu.SemaphoreType.DMA((2,)),
            pltpu.SemaphoreType.DMA((6,)),
        ],
        compiler_params=pltpu.CompilerParams(collective_id=0),
    )(x16, wq16, k2, v2, wo16)


# device time: 58540 ns/iter; 3.4636x vs baseline; 1.0911x over previous
import jax
import jax.numpy as jnp
from jax import lax
from jax.experimental import pallas as pl
from jax.experimental.pallas import tpu as pltpu

N_DEV = 4
B, SQ, SKV, H, D = 2, 512, 512, 8, 64
DP = 128
L_COL = D
M_COL = D + 1
BH = B * H
DM = 768
HD = H * D
HALO = 128
G = 32
T = 256
BAND = 384
NEG = -1e9


def kernel(x, Wq, K_ext, V_ext, Wo):
    bf16 = jnp.bfloat16
    k2 = jnp.transpose(K_ext, (0, 2, 1, 3)).reshape(BH, SKV, D).astype(bf16)
    k2 = jnp.concatenate([k2, jnp.zeros((BH, SKV, DP - D), bf16)], axis=2)
    v2 = jnp.transpose(V_ext, (0, 2, 1, 3)).reshape(BH, SKV, D).astype(bf16)
    v2 = jnp.concatenate([v2, jnp.ones((BH, SKV, 1), bf16),
                          jnp.zeros((BH, SKV, DP - D - 1), bf16)], axis=2)
    x16 = x.astype(bf16)
    wq16 = (Wq * 0.125).astype(bf16)
    wo16 = Wo.astype(bf16)

    def body(x_ref, wq_ref, k_ref, v_ref, wo_ref, out_ref,
             q_all, ctx_all,
             khalo_l, vhalo_l, khalo_r, vhalo_r, kglob, vglob,
             bias_band, bias_halo, bias_g,
             q32_buf, c32_send, c32_recv,
             hsend, hrecv, gsend, grecv, qsend, qrecv, psend, precv):
        my = lax.axis_index("i")
        left = my - 1
        right = my + 1
        has_left = my > 0
        has_right = my < N_DEV - 1
        is0 = my == 0

        q_all[...] = jnp.zeros_like(q_all)
        q32_buf[...] = jnp.zeros_like(q32_buf)
        khalo_l[...] = jnp.zeros_like(khalo_l)
        vhalo_l[...] = jnp.zeros_like(vhalo_l)
        khalo_r[...] = jnp.zeros_like(khalo_r)
        vhalo_r[...] = jnp.zeros_like(vhalo_r)

        n_partners = (has_left.astype(jnp.int32) + has_right.astype(jnp.int32)
                      + jnp.where(is0, 2, 0) + jnp.where(my >= 2, 1, 0))
        barrier_sem = pltpu.get_barrier_semaphore()

        def _signal(sem, dev):
            pl.semaphore_signal(sem, inc=1, device_id=(dev,),
                                device_id_type=pl.DeviceIdType.MESH)

        @pl.when(has_left)
        def _():
            _signal(barrier_sem, left)

        @pl.when(has_right)
        def _():
            _signal(barrier_sem, right)

        @pl.when(is0)
        def _():
            _signal(barrier_sem, 2)
            _signal(barrier_sem, 3)

        @pl.when(my >= 2)
        def _():
            _signal(barrier_sem, 0)

        pl.semaphore_wait(barrier_sem, n_partners)

        halo_r_k = pltpu.make_async_remote_copy(
            src_ref=k_ref.at[:, SKV - HALO:SKV, :], dst_ref=khalo_l,
            send_sem=hsend.at[0], recv_sem=hrecv.at[0],
            device_id=(right,), device_id_type=pl.DeviceIdType.MESH)
        halo_r_v = pltpu.make_async_remote_copy(
            src_ref=v_ref.at[:, SKV - HALO:SKV, :], dst_ref=vhalo_l,
            send_sem=hsend.at[1], recv_sem=hrecv.at[1],
            device_id=(right,), device_id_type=pl.DeviceIdType.MESH)
        halo_l_k = pltpu.make_async_remote_copy(
            src_ref=k_ref.at[:, 0:HALO, :], dst_ref=khalo_r,
            send_sem=hsend.at[2], recv_sem=hrecv.at[2],
            device_id=(left,), device_id_type=pl.DeviceIdType.MESH)
        halo_l_v = pltpu.make_async_remote_copy(
            src_ref=v_ref.at[:, 0:HALO, :], dst_ref=vhalo_r,
            send_sem=hsend.at[3], recv_sem=hrecv.at[3],
            device_id=(left,), device_id_type=pl.DeviceIdType.MESH)
        glob_k = [pltpu.make_async_remote_copy(
            src_ref=k_ref.at[:, 0:G, :], dst_ref=kglob,
            send_sem=gsend.at[t - 1], recv_sem=grecv.at[0],
            device_id=(t,), device_id_type=pl.DeviceIdType.MESH)
            for t in (1, 2, 3)]
        glob_v = [pltpu.make_async_remote_copy(
            src_ref=v_ref.at[:, 0:G, :], dst_ref=vglob,
            send_sem=gsend.at[t + 2], recv_sem=grecv.at[1],
            device_id=(t,), device_id_type=pl.DeviceIdType.MESH)
            for t in (1, 2, 3)]
        q32_bcast = [pltpu.make_async_remote_copy(
            src_ref=q32_buf, dst_ref=q32_buf,
            send_sem=qsend.at[t - 1], recv_sem=qrecv.at[0],
            device_id=(t,), device_id_type=pl.DeviceIdType.MESH)
            for t in (1, 2, 3)]
        part_c = pltpu.make_async_remote_copy(
            src_ref=c32_send, dst_ref=c32_recv.at[my - 1],
            send_sem=psend.at[0], recv_sem=precv.at[my - 1],
            device_id=(0,), device_id_type=pl.DeviceIdType.MESH)
        part_c_recv = [pltpu.make_async_remote_copy(
            src_ref=c32_send, dst_ref=c32_recv.at[r - 1],
            send_sem=psend.at[0], recv_sem=precv.at[r - 1],
            device_id=(0,), device_id_type=pl.DeviceIdType.MESH)
            for r in (1, 2, 3)]

        @pl.when(has_right)
        def _():
            halo_r_k.start()
            halo_r_v.start()

        @pl.when(has_left)
        def _():
            halo_l_k.start()
            halo_l_v.start()

        @pl.when(is0)
        def _():
            for t in range(3):
                glob_k[t].start()
                glob_v[t].start()
            for b in range(B):
                q32_b = jnp.dot(x_ref[b, 0:G, :], wq_ref[...],
                                preferred_element_type=jnp.float32)
                for h in range(H):
                    q32_buf[b * H + h, :, 0:D] = q32_b[
                        :, h * D:(h + 1) * D].astype(jnp.bfloat16)
            for t in range(3):
                q32_bcast[t].start()
            kglob[...] = k_ref[:, 0:G, :]
            vglob[...] = v_ref[:, 0:G, :]

        for b in range(B):
            q_b = jnp.dot(x_ref[b], wq_ref[...],
                          preferred_element_type=jnp.float32)
            for h in range(H):
                q_all[b * H + h, :, 0:D] = q_b[:, h * D:(h + 1) * D].astype(
                    jnp.bfloat16)

        ib = lax.broadcasted_iota(jnp.int32, (T, BAND), 0)
        jb = lax.broadcasted_iota(jnp.int32, (T, BAND), 1)
        ih = lax.broadcasted_iota(jnp.int32, (T, HALO), 0)
        jh = lax.broadcasted_iota(jnp.int32, (T, HALO), 1)
        ig = lax.broadcasted_iota(jnp.int32, (T, G), 0)
        jg = lax.broadcasted_iota(jnp.int32, (T, G), 1)
        zero = jnp.float32(0)
        for t in range(2):
            a0 = 0 if t == 0 else SKV - BAND
            bias_band[t] = jnp.where(
                jnp.abs((ib + t * T) - (jb + a0)) <= HALO, zero, NEG)
        bias_halo[0] = jnp.where(has_left & (ih <= jh), zero, NEG)
        bias_halo[1] = jnp.where(has_right & (ih >= HALO + jh), zero, NEG)
        bias_g[...] = jnp.where((~is0) | (ig > HALO + jg), zero, NEG)

        @pl.when(~is0)
        def _():
            q32_bcast[0].wait_recv()

        def part_step(bh, _):
            s32 = lax.dot_general(q32_buf[bh], k_ref[bh],
                                  (((1,), (1,)), ((), ())),
                                  preferred_element_type=jnp.float32)
            m32 = jnp.max(s32, axis=1, keepdims=True)
            w32 = jnp.exp(s32 - m32).astype(jnp.bfloat16)
            ce = lax.dot_general(w32, v_ref[bh], (((1,), (0,)), ((), ())),
                                 preferred_element_type=jnp.float32)
            c32_send[bh] = ce
            c32_send[bh, :, M_COL:M_COL + 1] = m32
            return 0

        lax.fori_loop(0, BH, part_step, 0)

        @pl.when(~is0)
        def _():
            part_c.start()

        @pl.when(has_left)
        def _():
            halo_r_k.wait_recv()
            halo_r_v.wait_recv()

        @pl.when(has_right)
        def _():
            halo_l_k.wait_recv()
            halo_l_v.wait_recv()

        @pl.when(~is0)
        def _():
            glob_k[0].wait_recv()
            glob_v[0].wait_recv()

        def bh_step(bh, _):
            q = q_all[bh]

            for t in range(2):
                q_t = q[t * T:(t + 1) * T, :]
                a0 = 0 if t == 0 else SKV - BAND
                s = lax.dot_general(q_t, k_ref[bh, a0:a0 + BAND, :],
                                    (((1,), (1,)), ((), ())),
                                    preferred_element_type=jnp.float32)
                w = jnp.exp(s + bias_band[t]).astype(jnp.bfloat16)
                acc = lax.dot_general(w, v_ref[bh, a0:a0 + BAND, :],
                                      (((1,), (0,)), ((), ())),
                                      preferred_element_type=jnp.float32)
                kh = khalo_l if t == 0 else khalo_r
                vh = vhalo_l if t == 0 else vhalo_r
                s = lax.dot_general(q_t, kh[bh], (((1,), (1,)), ((), ())),
                                    preferred_element_type=jnp.float32)
                w = jnp.exp(s + bias_halo[t]).astype(jnp.bfloat16)
                acc = acc + lax.dot_general(w, vh[bh], (((1,), (0,)), ((), ())),
                                            preferred_element_type=jnp.float32)
                s = lax.dot_general(q_t, kglob[bh], (((1,), (1,)), ((), ())),
                                    preferred_element_type=jnp.float32)
                if t == 0:
                    s = s + bias_g[...]
                w = jnp.exp(s).astype(jnp.bfloat16)
                acc = acc + lax.dot_general(w, vglob[bh],
                                            (((1,), (0,)), ((), ())),
                                            preferred_element_type=jnp.float32)
                ctx_all[bh, t * T:(t + 1) * T, :] = (
                    acc[:, 0:D] / acc[:, L_COL:L_COL + 1]).astype(jnp.bfloat16)
            return 0

        lax.fori_loop(0, BH, bh_step, 0)

        @pl.when(is0)
        def _():
            for r in range(3):
                part_c_recv[r].wait_recv()
            ms = [c32_send[:, :, M_COL:M_COL + 1]] + [
                c32_recv[r, :, :, M_COL:M_COL + 1] for r in range(3)]
            ls = [c32_send[:, :, L_COL:L_COL + 1]] + [
                c32_recv[r, :, :, L_COL:L_COL + 1] for r in range(3)]
            cs = [c32_send[:, :, 0:D]] + [c32_recv[r, :, :, 0:D]
                                          for r in range(3)]
            M = jnp.maximum(jnp.maximum(ms[0], ms[1]),
                            jnp.maximum(ms[2], ms[3]))
            L = jnp.zeros_like(ls[0])
            C = jnp.zeros_like(cs[0])
            for s in range(4):
                a = jnp.exp(ms[s] - M)
                L = L + a * ls[s]
                C = C + a * cs[s]
            ctx_all[:, 0:G, :] = (C / L).astype(jnp.bfloat16)

        for b in range(B):
            acc = jnp.zeros((SQ, DM), jnp.float32)
            for h in range(H):
                acc = acc + jnp.dot(ctx_all[b * H + h],
                                    wo_ref[h * D:(h + 1) * D, :],
                                    preferred_element_type=jnp.float32)
            out_ref[b] = acc

        @pl.when(has_right)
        def _():
            halo_r_k.wait_send()
            halo_r_v.wait_send()

        @pl.when(has_left)
        def _():
            halo_l_k.wait_send()
            halo_l_v.wait_send()

        @pl.when(is0)
        def _():
            for t in range(3):
                glob_k[t].wait_send()
                glob_v[t].wait_send()
                q32_bcast[t].wait_send()

        @pl.when(~is0)
        def _():
            part_c.wait_send()

        def _second(second_barrier):
            @pl.when(has_left)
            def _():
                _signal(second_barrier, left)

            @pl.when(has_right)
            def _():
                _signal(second_barrier, right)

            @pl.when(is0)
            def _():
                _signal(second_barrier, 2)
                _signal(second_barrier, 3)

            @pl.when(my >= 2)
            def _():
                _signal(second_barrier, 0)

            pl.semaphore_wait(second_barrier, n_partners)

        pl.run_scoped(_second, second_barrier=pltpu.SemaphoreType.REGULAR)

    return pl.pallas_call(
        body,
        out_shape=jax.ShapeDtypeStruct((B, SQ, DM), jnp.float32),
        in_specs=[pl.BlockSpec(memory_space=pltpu.VMEM)] * 5,
        out_specs=pl.BlockSpec(memory_space=pltpu.VMEM),
        scratch_shapes=[
            pltpu.VMEM((BH, SQ, DP), jnp.bfloat16),
            pltpu.VMEM((BH, SQ, D), jnp.bfloat16),
            pltpu.VMEM((BH, HALO, DP), jnp.bfloat16),
            pltpu.VMEM((BH, HALO, DP), jnp.bfloat16),
            pltpu.VMEM((BH, HALO, DP), jnp.bfloat16),
            pltpu.VMEM((BH, HALO, DP), jnp.bfloat16),
            pltpu.VMEM((BH, G, DP), jnp.bfloat16),
            pltpu.VMEM((BH, G, DP), jnp.bfloat16),
            pltpu.VMEM((2, T, BAND), jnp.float32),
            pltpu.VMEM((2, T, HALO), jnp.float32),
            pltpu.VMEM((T, G), jnp.float32),
            pltpu.VMEM((BH, G, DP), jnp.bfloat16),
            pltpu.VMEM((BH, G, DP), jnp.float32),
            pltpu.VMEM((3, BH, G, DP), jnp.float32),
            pltpu.SemaphoreType.DMA((4,)),
            pltpu.SemaphoreType.DMA((4,)),
            pltpu.SemaphoreType.DMA((6,)),
            pltpu.SemaphoreType.DMA((2,)),
            pltpu.SemaphoreType.DMA((3,)),
            pltpu.SemaphoreType.DMA((1,)),
            pltpu.SemaphoreType.DMA((1,)),
            pltpu.SemaphoreType.DMA((3,)),
        ],
        compiler_params=pltpu.CompilerParams(collective_id=0),
    )(x16, wq16, k2, v2, wo16)
